# baseline (speedup 1.0000x reference)
import functools

import jax
import jax.numpy as jnp
from jax import lax
from jax.experimental import pallas as pl
from jax.experimental.pallas import tpu as pltpu
from jax.experimental.pallas import tpu_sc as plsc

DIM = 64
NUM_CORES = 2
NUM_SUBCORES = 16
NW = NUM_CORES * NUM_SUBCORES


@functools.partial(jax.jit, static_argnames=("rpc", "nbuf"))
def _emb_lookup(x, table, rpc=8, nbuf=2):
    s0, s1 = x.shape
    rows_w = s0 // NW
    n_chunks = rows_w // rpc
    assert n_chunks % nbuf == 0 and n_chunks > nbuf

    mesh = plsc.VectorSubcoreMesh(core_axis_name="c", subcore_axis_name="s")

    @functools.partial(
        pl.kernel,
        mesh=mesh,
        out_type=jax.ShapeDtypeStruct((s0, s1, DIM), jnp.float32),
        scratch_types=[
            pltpu.VMEM((rows_w, s1), jnp.int32),
            pltpu.VMEM((nbuf, rpc, s1, DIM), jnp.float32),
            [pltpu.SemaphoreType.DMA] * nbuf,
            [pltpu.SemaphoreType.DMA] * nbuf,
        ],
        compiler_params=pltpu.CompilerParams(use_tc_tiling_on_sc=False),
    )
    def k(idx_hbm, table_hbm, out_hbm, idx_v, rows_v, gsems, ssems):
        wid = lax.axis_index("s") * NUM_CORES + lax.axis_index("c")
        row_base = wid * rows_w

        def sub_gather(g, b, j):
            return pltpu.make_async_copy(
                table_hbm.at[idx_v.at[g * rpc + j]],
                rows_v.at[b, j],
                gsems[b],
            )

        def gather_start(g, b):
            for j in range(rpc):
                sub_gather(g, b, j).start()

        def gather_wait(g, b):
            for j in range(rpc):
                sub_gather(g, b, j).wait()

        def store(g, b):
            return pltpu.make_async_copy(
                rows_v.at[b],
                out_hbm.at[pl.ds(row_base + g * rpc, rpc)],
                ssems[b],
            )

        pltpu.sync_copy(idx_hbm.at[pl.ds(row_base, rows_w)], idx_v)

        for b in range(nbuf):
            gather_start(b, b)

        @pl.loop(0, n_chunks - nbuf, step=nbuf)
        def _(g0):
            for b in range(nbuf):
                g = g0 + b
                gather_wait(g, b)
                store(g, b).start()
                store(g, b).wait()
                gather_start(g + nbuf, b)

        for b in range(nbuf):
            g = n_chunks - nbuf + b
            gather_wait(g, b)
            store(g, b).start()
        for b in range(nbuf):
            g = n_chunks - nbuf + b
            store(g, b).wait()

    return k(x, table)


def kernel(x, table):
    return _emb_lookup(x.astype(jnp.int32), table)


# speedup vs baseline: 1.1534x; 1.1534x over previous
import functools

import jax
import jax.numpy as jnp
from jax import lax
from jax.experimental import pallas as pl
from jax.experimental.pallas import tpu as pltpu
from jax.experimental.pallas import tpu_sc as plsc

DIM = 64
PAD_DIM = 128
NUM_CORES = 2
NUM_SUBCORES = 16
NW = NUM_CORES * NUM_SUBCORES
TBLK = 4096


def _transpose_body(t_ref, o_ref):
    o_ref[:, :DIM] = jnp.transpose(t_ref[...], (1, 0))


def _relayout_table(tT):
    V = tT.shape[1]
    nblk = (V + TBLK - 1) // TBLK
    return pl.pallas_call(
        _transpose_body,
        grid=(nblk,),
        in_specs=[pl.BlockSpec((DIM, TBLK), lambda j: (0, j))],
        out_specs=pl.BlockSpec((TBLK, PAD_DIM), lambda j: (j, 0)),
        out_shape=jax.ShapeDtypeStruct((nblk * TBLK, PAD_DIM), jnp.float32),
    )(tT)


@functools.partial(jax.jit, static_argnames=("rpc", "nbuf"))
def _emb_lookup(x, tT, rpc=8, nbuf=2):
    s0, s1 = x.shape
    rows_w = s0 // NW
    n_chunks = rows_w // rpc
    assert n_chunks % nbuf == 0 and n_chunks > nbuf

    table_lin = _relayout_table(tT)
    mesh = plsc.VectorSubcoreMesh(core_axis_name="c", subcore_axis_name="s")

    @functools.partial(
        pl.kernel,
        mesh=mesh,
        out_type=jax.ShapeDtypeStruct((s0, s1, DIM), jnp.float32),
        scratch_types=[
            pltpu.VMEM((rows_w, s1), jnp.int32),
            pltpu.VMEM((nbuf, rpc, s1, PAD_DIM), jnp.float32),
            [pltpu.SemaphoreType.DMA] * nbuf,
            [pltpu.SemaphoreType.DMA] * nbuf,
        ],
        compiler_params=pltpu.CompilerParams(use_tc_tiling_on_sc=False),
    )
    def k(idx_hbm, table_hbm, out_hbm, idx_v, rows_v, gsems, ssems):
        wid = lax.axis_index("s") * NUM_CORES + lax.axis_index("c")
        row_base = wid * rows_w

        def sub_gather(g, b, j):
            return pltpu.make_async_copy(
                table_hbm.at[idx_v.at[g * rpc + j]],
                rows_v.at[b, j],
                gsems[b],
            )

        def gather_start(g, b):
            for j in range(rpc):
                sub_gather(g, b, j).start()

        def gather_wait(g, b):
            for j in range(rpc):
                sub_gather(g, b, j).wait()

        def store(g, b):
            return pltpu.make_async_copy(
                rows_v.at[b, :, :, pl.ds(0, DIM)],
                out_hbm.at[pl.ds(row_base + g * rpc, rpc)],
                ssems[b],
            )

        pltpu.sync_copy(idx_hbm.at[pl.ds(row_base, rows_w)], idx_v)

        for b in range(nbuf):
            gather_start(b, b)

        @pl.loop(0, n_chunks - nbuf, step=nbuf)
        def _(g0):
            for b in range(nbuf):
                g = g0 + b
                gather_wait(g, b)
                store(g, b).start()
                store(g, b).wait()
                gather_start(g + nbuf, b)

        for b in range(nbuf):
            g = n_chunks - nbuf + b
            gather_wait(g, b)
            store(g, b).start()
        for b in range(nbuf):
            g = n_chunks - nbuf + b
            store(g, b).wait()

    return k(x, table_lin)


def kernel(x, table):
    return _emb_lookup(x.astype(jnp.int32), jnp.swapaxes(table, 0, 1))


# speedup vs baseline: 1.1534x; 1.0000x over previous
import functools

import jax
import jax.numpy as jnp
from jax import lax
from jax.experimental import pallas as pl
from jax.experimental.pallas import tpu as pltpu
from jax.experimental.pallas import tpu_sc as plsc

DIM = 64
PAD_DIM = 128
NUM_CORES = 2
NUM_SUBCORES = 16
NW = NUM_CORES * NUM_SUBCORES
TBLK = 4096


def _transpose_body(t_ref, o_ref):
    tr = jnp.transpose(t_ref[...], (1, 0)).reshape(TBLK // 2, 2, DIM)
    o_ref[:, :DIM] = tr[:, 0, :]
    o_ref[:, DIM:] = tr[:, 1, :]


def _relayout_table(tT):
    V = tT.shape[1]
    nblk = (V + TBLK - 1) // TBLK
    return pl.pallas_call(
        _transpose_body,
        grid=(nblk,),
        in_specs=[pl.BlockSpec((DIM, TBLK), lambda j: (0, j))],
        out_specs=pl.BlockSpec((TBLK // 2, PAD_DIM), lambda j: (j, 0)),
        out_shape=jax.ShapeDtypeStruct((nblk * TBLK // 2, PAD_DIM), jnp.float32),
    )(tT)


@functools.partial(jax.jit, static_argnames=("rpc", "nbuf"))
def _emb_lookup(x, tT, rpc=8, nbuf=2):
    s0, s1 = x.shape
    rows_w = s0 // NW
    n_chunks = rows_w // rpc
    assert n_chunks % nbuf == 0 and n_chunks > nbuf

    table_pack = _relayout_table(tT)
    table_lin = jnp.reshape(table_pack, (table_pack.shape[0] * 2, DIM))
    mesh = plsc.VectorSubcoreMesh(core_axis_name="c", subcore_axis_name="s")

    @functools.partial(
        pl.kernel,
        mesh=mesh,
        out_type=jax.ShapeDtypeStruct((s0, s1, DIM), jnp.float32),
        scratch_types=[
            pltpu.VMEM((rows_w, s1), jnp.int32),
            pltpu.VMEM((nbuf, rpc, s1, DIM), jnp.float32),
            [pltpu.SemaphoreType.DMA] * nbuf,
            [pltpu.SemaphoreType.DMA] * nbuf,
        ],
        compiler_params=pltpu.CompilerParams(use_tc_tiling_on_sc=False),
    )
    def k(idx_hbm, table_hbm, out_hbm, idx_v, rows_v, gsems, ssems):
        wid = lax.axis_index("s") * NUM_CORES + lax.axis_index("c")
        row_base = wid * rows_w

        def sub_gather(g, b, j):
            return pltpu.make_async_copy(
                table_hbm.at[idx_v.at[g * rpc + j]],
                rows_v.at[b, j],
                gsems[b],
            )

        def gather_start(g, b):
            for j in range(rpc):
                sub_gather(g, b, j).start()

        def gather_wait(g, b):
            for j in range(rpc):
                sub_gather(g, b, j).wait()

        def store(g, b):
            return pltpu.make_async_copy(
                rows_v.at[b],
                out_hbm.at[pl.ds(row_base + g * rpc, rpc)],
                ssems[b],
            )

        pltpu.sync_copy(idx_hbm.at[pl.ds(row_base, rows_w)], idx_v)

        for b in range(nbuf):
            gather_start(b, b)

        @pl.loop(0, n_chunks - nbuf, step=nbuf)
        def _(g0):
            for b in range(nbuf):
                g = g0 + b
                gather_wait(g, b)
                store(g, b).start()
                store(g, b).wait()
                gather_start(g + nbuf, b)

        for b in range(nbuf):
            g = n_chunks - nbuf + b
            gather_wait(g, b)
            store(g, b).start()
        for b in range(nbuf):
            g = n_chunks - nbuf + b
            store(g, b).wait()

    return k(x, table_lin)


def kernel(x, table):
    return _emb_lookup(x.astype(jnp.int32), jnp.swapaxes(table, 0, 1))
